# baseline (speedup 1.0000x reference)
import functools

import jax
import jax.numpy as jnp
from jax.experimental import pallas as pl
from jax.experimental.pallas import tpu as pltpu

EPS = 1e-7
LABEL_SMOOTHING = 0.1

NCH = 4


def _half(mt, pred):
    cls = jnp.min(mt, axis=(1, 2), keepdims=True)
    valid = cls < 80
    sc = jnp.where(valid, cls, 0)
    lane = jax.lax.broadcasted_iota(jnp.int32, pred.shape, 3)
    chosen_pred = jnp.sum(jnp.where(lane == sc[..., None], pred, 0.0), axis=-1)
    chosen_true = (mt == sc).astype(jnp.float32)
    y = (1.0 - LABEL_SMOOTHING) * chosen_true + LABEL_SMOOTHING / 2.0
    loss = -(y * jnp.log(chosen_pred + EPS)
             + (1.0 - y) * jnp.log(1.0 - chosen_pred + EPS))
    molded = jnp.where(valid[:, 0, 0], jnp.mean(loss, axis=(1, 2)), 0.0)
    return jnp.sum(molded), jnp.sum((molded != 0.0).astype(jnp.float32))


def _body(mt_ref, mpa_ref, mpb_ref, out_ref, acc_ref):
    b = pl.program_id(0)
    k = pl.program_id(1)

    @pl.when(k == 0)
    def _init():
        acc_ref[0] = 0.0
        acc_ref[1] = 0.0

    h = NCH // 2
    sa, ca = _half(mt_ref[0, :h], mpa_ref[0])
    sb, cb = _half(mt_ref[0, h:], mpb_ref[0])
    acc_ref[0] += sa + sb
    acc_ref[1] += ca + cb

    @pl.when(k == pl.num_programs(1) - 1)
    def _fin():
        out_ref[b] = acc_ref[0] / (acc_ref[1] + 1.0)


@jax.jit
def kernel(mask_true, mask_pred):
    B, N, H, W = mask_true.shape
    C = mask_pred.shape[-1]
    h = NCH // 2
    out = pl.pallas_call(
        _body,
        grid=(B, N // NCH),
        in_specs=[
            pl.BlockSpec((1, NCH, H, W), lambda b, k: (b, k, 0, 0)),
            pl.BlockSpec((1, h, H, W, C), lambda b, k: (b, 2 * k, 0, 0, 0)),
            pl.BlockSpec((1, h, H, W, C), lambda b, k: (b, 2 * k + 1, 0, 0, 0)),
        ],
        out_specs=pl.BlockSpec(memory_space=pltpu.SMEM),
        out_shape=jax.ShapeDtypeStruct((B,), jnp.float32),
        scratch_shapes=[pltpu.SMEM((2,), jnp.float32)],
    )(mask_true, mask_pred, mask_pred)
    return out


# speedup vs baseline: 1.0821x; 1.0821x over previous
import functools

import jax
import jax.numpy as jnp
from jax.experimental import pallas as pl
from jax.experimental.pallas import tpu as pltpu

EPS = 1e-7
LABEL_SMOOTHING = 0.1

NCH = 4


def _body(mt_ref, mp_ref, out_ref, acc_ref):
    b = pl.program_id(0)
    k = pl.program_id(1)

    @pl.when(k == 0)
    def _init():
        acc_ref[0] = 0.0
        acc_ref[1] = 0.0

    mt = mt_ref[0]
    cls = jnp.min(mt, axis=(1, 2), keepdims=True)
    valid = cls < 80
    sc = jnp.where(valid, cls, 0)

    pred = mp_ref[0]
    lane = jax.lax.broadcasted_iota(jnp.int32, pred.shape, 3)
    chosen_pred = jnp.sum(jnp.where(lane == sc[..., None], pred, 0.0), axis=-1)
    chosen_true = (mt == sc).astype(jnp.float32)

    y = (1.0 - LABEL_SMOOTHING) * chosen_true + LABEL_SMOOTHING / 2.0
    loss = -(y * jnp.log(chosen_pred + EPS)
             + (1.0 - y) * jnp.log(1.0 - chosen_pred + EPS))
    molded = jnp.where(valid[:, 0, 0], jnp.mean(loss, axis=(1, 2)), 0.0)
    acc_ref[0] += jnp.sum(molded)
    acc_ref[1] += jnp.sum((molded != 0.0).astype(jnp.float32))

    @pl.when(k == pl.num_programs(1) - 1)
    def _fin():
        out_ref[b] = acc_ref[0] / (acc_ref[1] + 1.0)


@jax.jit
def kernel(mask_true, mask_pred):
    B, N, H, W = mask_true.shape
    C = mask_pred.shape[-1]
    out = pl.pallas_call(
        _body,
        grid=(B, N // NCH),
        in_specs=[
            pl.BlockSpec((1, NCH, H, W), lambda b, k: (b, k, 0, 0)),
            pl.BlockSpec((1, NCH, H, W, C), lambda b, k: (b, k, 0, 0, 0)),
        ],
        out_specs=pl.BlockSpec(memory_space=pltpu.SMEM),
        out_shape=jax.ShapeDtypeStruct((B,), jnp.float32),
        scratch_shapes=[pltpu.SMEM((2,), jnp.float32)],
    )(mask_true, mask_pred)
    return out
